# baseline (speedup 1.0000x reference)
import jax
import jax.numpy as jnp
import numpy as np
from jax import lax
from jax.experimental import pallas as pl
from jax.experimental.pallas import tpu as pltpu
from jax.experimental.pallas import tpu_sc as plsc


def kernel(frames):
    C, T, H, W = frames.shape
    S = T // 4

    idx = np.linspace(0.0, T - 1, S).astype(np.int64)
    jt = (np.arange(T) * (S - 1) + (T - 2)) // (T - 1)
    assert np.array_equal(jt, np.searchsorted(idx, np.arange(T)))
    assert np.all(np.diff(idx) > 0)

    N = C * T
    NC, NS = 2, 16
    NW = NC * NS
    assert N % NW == 0
    RPW = N // NW

    mesh = plsc.VectorSubcoreMesh(
        core_axis_name="c", subcore_axis_name="s")

    CPF = 4
    SLOTS = 4
    HC = H // CPF
    assert H % CPF == 0 and HC % 8 == 0
    NCH = RPW * CPF

    def body(x_hbm, fast_hbm, slow_hbm, buf, insem, fsem, ssem):
        wid = lax.axis_index("s") * NC + lax.axis_index("c")

        def info(k):
            r = wid * RPW + (k // CPF)
            t = lax.rem(r, T)
            ch = lax.div(r, T)
            j0 = (t * (S - 1) + (T - 2)) // (T - 1)
            j1 = ((t + 1) * (S - 1) + (T - 2)) // (T - 1)
            issel = jnp.logical_or(t == T - 1, j1 != j0)
            h0 = (k % CPF) * HC
            return ch, t, j0, issel, h0

        def in_cp(k, s):
            ch, t, _, _, h0 = info(k)
            return pltpu.make_async_copy(
                x_hbm.at[ch, t, pl.ds(h0, HC)], buf.at[s], insem.at[s])

        def fast_cp(k, s):
            ch, t, _, _, h0 = info(k)
            return pltpu.make_async_copy(
                buf.at[s], fast_hbm.at[ch, t, pl.ds(h0, HC)], fsem.at[s])

        def slow_cp(k, s):
            ch, _, j0, _, h0 = info(k)
            return pltpu.make_async_copy(
                buf.at[s], slow_hbm.at[ch, j0, pl.ds(h0, HC)], ssem.at[s])

        def out_wait(k, s):
            fast_cp(k, s).wait()
            _, _, _, issel, _ = info(k)

            @pl.when(issel)
            def _():
                slow_cp(k, s).wait()

        for k in range(SLOTS):
            in_cp(k, k).start()
        for k in range(NCH):
            s = k % SLOTS
            if k >= 1:
                p = k - 1
                if p + SLOTS < NCH:
                    out_wait(p, p % SLOTS)
                    in_cp(p + SLOTS, p % SLOTS).start()
            in_cp(k, s).wait()
            fast_cp(k, s).start()
            _, _, _, issel, _ = info(k)

            @pl.when(issel)
            def _():
                slow_cp(k, s).start()

        for k in range(max(NCH - SLOTS, 0), NCH):
            out_wait(k, k % SLOTS)

    run = pl.kernel(
        body,
        out_type=[
            jax.ShapeDtypeStruct((C, T, H, W), frames.dtype),
            jax.ShapeDtypeStruct((C, S, H, W), frames.dtype),
        ],
        mesh=mesh,
        scratch_types=[
            pltpu.VMEM((SLOTS, HC, W), frames.dtype),
            pltpu.SemaphoreType.DMA((SLOTS,)),
            pltpu.SemaphoreType.DMA((SLOTS,)),
            pltpu.SemaphoreType.DMA((SLOTS,)),
        ],
    )
    fast, slow = run(frames)
    return (slow, fast)


# speedup vs baseline: 1.0159x; 1.0159x over previous
import jax
import jax.numpy as jnp
import numpy as np
from jax import lax
from jax.experimental import pallas as pl
from jax.experimental.pallas import tpu as pltpu
from jax.experimental.pallas import tpu_sc as plsc


def kernel(frames):
    C, T, H, W = frames.shape
    S = T // 4

    idx = np.linspace(0.0, T - 1, S).astype(np.int64)
    assert np.array_equal(idx, (np.arange(S) * (T - 1)) // (S - 1))

    NSEL = C * S
    NC, NS = 2, 16
    NW = NC * NS

    assert NSEL * 2 == 3 * NW
    HB = H // 2
    assert HB % 8 == 0

    mesh = plsc.VectorSubcoreMesh(
        core_axis_name="c", subcore_axis_name="s")

    def sc_body(x_hbm, slow_hbm, buf, insem, outsem):
        wid = lax.axis_index("s") * NC + lax.axis_index("c")
        two = wid < NS

        def finfo(f):
            ch = lax.div(f, S)
            jj = lax.rem(f, S)
            t = (jj * (T - 1)) // (S - 1)
            return ch, t, jj

        def in_cp(f, half, s):
            ch, t, _ = finfo(f)
            return pltpu.make_async_copy(
                x_hbm.at[ch, t, pl.ds(half * HB, HB)], buf.at[s],
                insem.at[s])

        def out_cp(f, half, s):
            ch, _, jj = finfo(f)
            return pltpu.make_async_copy(
                buf.at[s], slow_hbm.at[ch, jj, pl.ds(half * HB, HB)],
                outsem.at[s])

        f0 = jnp.where(two, 2 * wid, wid + NS)
        f1 = f0 + 1

        in_cp(f0, 0, 0).start()
        in_cp(f0, 1, 1).start()
        in_cp(f0, 0, 0).wait()
        out_cp(f0, 0, 0).start()
        in_cp(f0, 1, 1).wait()
        out_cp(f0, 1, 1).start()

        @pl.when(two)
        def _():
            out_cp(f0, 0, 0).wait()
            in_cp(f1, 0, 0).start()
            out_cp(f0, 1, 1).wait()
            in_cp(f1, 1, 1).start()
            in_cp(f1, 0, 0).wait()
            out_cp(f1, 0, 0).start()
            in_cp(f1, 1, 1).wait()
            out_cp(f1, 1, 1).start()
            out_cp(f1, 0, 0).wait()
            out_cp(f1, 1, 1).wait()

        @pl.when(jnp.logical_not(two))
        def _():
            out_cp(f0, 0, 0).wait()
            out_cp(f0, 1, 1).wait()

    sc_run = pl.kernel(
        sc_body,
        out_type=jax.ShapeDtypeStruct((C, S, H, W), frames.dtype),
        mesh=mesh,
        scratch_types=[
            pltpu.VMEM((2, HB, W), frames.dtype),
            pltpu.SemaphoreType.DMA((2,)),
            pltpu.SemaphoreType.DMA((2,)),
        ],
    )

    GF = 4
    assert T % GF == 0
    NG = C * (T // GF)
    TSLOTS = 4

    def tc_body(x_hbm, fast_hbm, buf, insem, outsem):
        def grp(g):
            return g // (T // GF), (g % (T // GF)) * GF

        def in_cp(g, s):
            ch, t0 = grp(g)
            return pltpu.make_async_copy(
                x_hbm.at[ch, pl.ds(t0, GF)], buf.at[s], insem.at[s])

        def out_cp(g, s):
            ch, t0 = grp(g)
            return pltpu.make_async_copy(
                buf.at[s], fast_hbm.at[ch, pl.ds(t0, GF)], outsem.at[s])

        for g in range(TSLOTS):
            in_cp(g, g).start()
        for g in range(NG):
            s = g % TSLOTS
            if g >= 1:
                p = g - 1
                if p + TSLOTS < NG:
                    out_cp(p, p % TSLOTS).wait()
                    in_cp(p + TSLOTS, p % TSLOTS).start()
            in_cp(g, s).wait()
            out_cp(g, s).start()
        for g in range(max(NG - TSLOTS, 0), NG):
            out_cp(g, g % TSLOTS).wait()

    fast = pl.pallas_call(
        tc_body,
        in_specs=[pl.BlockSpec(memory_space=pltpu.MemorySpace.HBM)],
        out_specs=pl.BlockSpec(memory_space=pltpu.MemorySpace.HBM),
        out_shape=jax.ShapeDtypeStruct((C, T, H, W), frames.dtype),
        scratch_shapes=[
            pltpu.VMEM((TSLOTS, GF, H, W), frames.dtype),
            pltpu.SemaphoreType.DMA((TSLOTS,)),
            pltpu.SemaphoreType.DMA((TSLOTS,)),
        ],
    )(frames)

    slow = sc_run(frames)
    return (slow, fast)
